# baseline (speedup 1.0000x reference)
import functools

import jax
import jax.numpy as jnp
from jax import lax
from jax.experimental import pallas as pl
from jax.experimental.pallas import tpu as pltpu
from jax.experimental.pallas import tpu_sc as plsc

_NUM_BUCKETS = 1000
_B = 16384
_H = 50
_D = 64
_W = _D // 2
_L = 16
_NC = 2
_NS = 16
_NW = _NC * _NS
_BPW = _B // _NW
_CH = 128
_NCH = _BPW // _CH

_LO = [w if w < _L else _L + w for w in range(_W)]
_HI = [w + _L if w < _L else 2 * _L + w for w in range(_W)]


def _tec_body(idx_hbm, qtab_hbm, rtab_hbm, out_hbm,
              raw_v, qtab_v, rtab_v, qT_v, rT_v, out_v):
    wid = lax.axis_index("s") * _NC + lax.axis_index("c")

    pltpu.sync_copy(qtab_hbm, qtab_v)
    pltpu.sync_copy(rtab_hbm, rtab_v)

    inv_b = jnp.float32(1.0 / _NUM_BUCKETS)
    inv_h = jnp.float32(1.0 / _H)
    iota = lax.iota(jnp.int32, _L)
    zeros = jnp.zeros((_L,), jnp.float32)

    def g_body(g, _):
        base = wid * _BPW + g * _CH
        pltpu.sync_copy(idx_hbm.at[pl.ds(base * _H, _CH * _H)], raw_v)

        def qr_body(i, _):
            x = raw_v[pl.ds(i * _L, _L)]
            p = i * _L + iota
            b0 = (p.astype(jnp.float32) * inv_h).astype(jnp.int32)
            j0 = p - b0 * _H
            bb = (b0 + jnp.where(j0 >= _H, 1, 0) - jnp.where(j0 < 0, 1, 0))
            jj = p - bb * _H
            dst = jj * _CH + bb
            q0 = (x.astype(jnp.float32) * inv_b).astype(jnp.int32)
            r0 = x - q0 * _NUM_BUCKETS
            q = (q0 + jnp.where(r0 >= _NUM_BUCKETS, 1, 0)
                 - jnp.where(r0 < 0, 1, 0))
            plsc.store_scatter(qT_v, [dst], q)
            plsc.store_scatter(rT_v, [dst], x - q * _NUM_BUCKETS)
            return 0
        lax.fori_loop(0, _H * _CH // _L, qr_body, 0, unroll=4)

        def b16_body(t, _):
            off = t * _L
            rows = off + iota
            for wg in range(2):
                def j_body(j, acc):
                    jb = j * _CH + off
                    qv = qT_v[pl.ds(jb, _L)]
                    rv = rT_v[pl.ds(jb, _L)]
                    new = list(acc)
                    for w in range(wg * _L, wg * _L + _L):
                        gq = plsc.load_gather(qtab_v.at[w], [qv])
                        gr = plsc.load_gather(rtab_v.at[w], [rv])
                        t2 = (plsc.bitcast(gq, jnp.bfloat16)
                              + plsc.bitcast(gr, jnp.bfloat16))
                        a, b = plsc.unpack(
                            t2, format=plsc.PackFormat.INTERLEAVED)
                        s = w - wg * _L
                        new[2 * s] = new[2 * s] + a
                        new[2 * s + 1] = new[2 * s + 1] + b
                    return tuple(new)
                acc = lax.fori_loop(0, _H, j_body, (zeros,) * (2 * _L))
                for w in range(wg * _L, wg * _L + _L):
                    s = w - wg * _L
                    plsc.store_scatter(
                        out_v, [rows, jnp.full((_L,), _LO[w], jnp.int32)],
                        acc[2 * s])
                    plsc.store_scatter(
                        out_v, [rows, jnp.full((_L,), _HI[w], jnp.int32)],
                        acc[2 * s + 1])
            return 0
        lax.fori_loop(0, _CH // _L, b16_body, 0)

        pltpu.sync_copy(out_v, out_hbm.at[pl.ds(base, _CH)])
        return 0
    lax.fori_loop(0, _NCH, g_body, 0)


_mesh = plsc.VectorSubcoreMesh(core_axis_name="c", subcore_axis_name="s")

_qr_bag = functools.partial(
    pl.kernel,
    mesh=_mesh,
    out_type=jax.ShapeDtypeStruct((_B, _D), jnp.float32),
    scratch_types=[
        pltpu.VMEM((_CH * _H,), jnp.int32),
        pltpu.VMEM((_W, _NUM_BUCKETS), jnp.int32),
        pltpu.VMEM((_W, _NUM_BUCKETS), jnp.int32),
        pltpu.VMEM((_H * _CH,), jnp.int32),
        pltpu.VMEM((_H * _CH,), jnp.int32),
        pltpu.VMEM((_CH, _D), jnp.float32),
    ],
    compiler_params=pltpu.CompilerParams(use_tc_tiling_on_sc=False,
                                         needs_layout_passes=False),
)(_tec_body)


def _pack_table(w):
    u = lax.bitcast_convert_type(w.astype(jnp.bfloat16), jnp.uint16)
    u = u.astype(jnp.uint32)
    lo = jnp.concatenate([u[:, 0:16], u[:, 32:48]], axis=1)
    hi = jnp.concatenate([u[:, 16:32], u[:, 48:64]], axis=1)
    packed = lo | (hi << 16)
    return lax.bitcast_convert_type(packed, jnp.int32).T


def kernel(input_, quotient_embed_weight, remainder_embed_weight):
    idx = input_.astype(jnp.int32).reshape(-1)
    return _qr_bag(idx,
                   _pack_table(quotient_embed_weight),
                   _pack_table(remainder_embed_weight))


# speedup vs baseline: 1.1789x; 1.1789x over previous
import functools

import jax
import jax.numpy as jnp
from jax import lax
from jax.experimental import pallas as pl
from jax.experimental.pallas import tpu as pltpu
from jax.experimental.pallas import tpu_sc as plsc

_NUM_BUCKETS = 1000
_B = 16384
_H = 50
_D = 64
_W = _D // 2
_L = 16
_NC = 2
_NS = 16
_NW = _NC * _NS
_BPW = _B // _NW
_CH = 128
_NCH = _BPW // _CH

_LO = [w if w < _L else _L + w for w in range(_W)]
_HI = [w + _L if w < _L else 2 * _L + w for w in range(_W)]


def _tec_body(idxT_hbm, qtab_hbm, rtab_hbm, outT_hbm,
              qtab_v, rtab_v, qT_v, rT_v, out_v):
    wid = lax.axis_index("s") * _NC + lax.axis_index("c")

    pltpu.sync_copy(qtab_hbm, qtab_v)
    pltpu.sync_copy(rtab_hbm, rtab_v)

    inv = jnp.float32(1.0 / _NUM_BUCKETS)
    zeros = jnp.zeros((_L,), jnp.float32)
    iota = lax.iota(jnp.int32, _L)

    def g_body(g, _):
        base = wid * _BPW + g * _CH
        pltpu.sync_copy(idxT_hbm.at[:, pl.ds(base, _CH)], qT_v)

        def qr_body(i, _):
            row = lax.shift_right_logical(i, 3)
            col = (i & 7) * _L
            x = qT_v[row, pl.ds(col, _L)]
            q0 = (x.astype(jnp.float32) * inv).astype(jnp.int32)
            r0 = x - q0 * _NUM_BUCKETS
            q = (q0 + jnp.where(r0 >= _NUM_BUCKETS, 1, 0)
                 - jnp.where(r0 < 0, 1, 0))
            qT_v[row, pl.ds(col, _L)] = q
            rT_v[row, pl.ds(col, _L)] = x - q * _NUM_BUCKETS
            return 0
        lax.fori_loop(0, _H * (_CH // _L), qr_body, 0, unroll=4)

        def b16_body(t, _):
            off = t * _L
            rows = off + iota
            for wg in range(2):
                def j_body(j, acc):
                    qv = qT_v[j, pl.ds(off, _L)]
                    rv = rT_v[j, pl.ds(off, _L)]
                    new = list(acc)
                    for w in range(wg * _L, wg * _L + _L):
                        gq = plsc.load_gather(qtab_v.at[w], [qv])
                        gr = plsc.load_gather(rtab_v.at[w], [rv])
                        t = (plsc.bitcast(gq, jnp.bfloat16)
                             + plsc.bitcast(gr, jnp.bfloat16))
                        a, b = plsc.unpack(
                            t, format=plsc.PackFormat.INTERLEAVED)
                        s = w - wg * _L
                        new[2 * s] = new[2 * s] + a
                        new[2 * s + 1] = new[2 * s + 1] + b
                    return tuple(new)
                acc = lax.fori_loop(0, _H, j_body, (zeros,) * (2 * _L))
                for w in range(wg * _L, wg * _L + _L):
                    s = w - wg * _L
                    plsc.store_scatter(
                        out_v, [rows, jnp.full((_L,), _LO[w], jnp.int32)],
                        acc[2 * s])
                    plsc.store_scatter(
                        out_v, [rows, jnp.full((_L,), _HI[w], jnp.int32)],
                        acc[2 * s + 1])
            return 0
        lax.fori_loop(0, _CH // _L, b16_body, 0)

        pltpu.sync_copy(out_v, outT_hbm.at[pl.ds(base, _CH)])
        return 0
    lax.fori_loop(0, _NCH, g_body, 0)


_mesh = plsc.VectorSubcoreMesh(core_axis_name="c", subcore_axis_name="s")

_qr_bag = functools.partial(
    pl.kernel,
    mesh=_mesh,
    out_type=jax.ShapeDtypeStruct((_B, _D), jnp.float32),
    scratch_types=[
        pltpu.VMEM((_W, _NUM_BUCKETS), jnp.int32),
        pltpu.VMEM((_W, _NUM_BUCKETS), jnp.int32),
        pltpu.VMEM((_H, _CH), jnp.int32),
        pltpu.VMEM((_H, _CH), jnp.int32),
        pltpu.VMEM((_CH, _D), jnp.float32),
    ],
    compiler_params=pltpu.CompilerParams(use_tc_tiling_on_sc=False,
                                         needs_layout_passes=False),
)(_tec_body)


def _pack_table(w):
    u = lax.bitcast_convert_type(w.astype(jnp.bfloat16), jnp.uint16)
    u = u.astype(jnp.uint32)
    lo = jnp.concatenate([u[:, 0:16], u[:, 32:48]], axis=1)
    hi = jnp.concatenate([u[:, 16:32], u[:, 48:64]], axis=1)
    packed = lo | (hi << 16)
    return lax.bitcast_convert_type(packed, jnp.int32).T


def kernel(input_, quotient_embed_weight, remainder_embed_weight):
    idx_t = input_.astype(jnp.int32).T
    return _qr_bag(idx_t,
                   _pack_table(quotient_embed_weight),
                   _pack_table(remainder_embed_weight))


# speedup vs baseline: 1.3160x; 1.1163x over previous
import functools

import jax
import jax.numpy as jnp
from jax import lax
from jax.experimental import pallas as pl
from jax.experimental.pallas import tpu as pltpu
from jax.experimental.pallas import tpu_sc as plsc

_NUM_BUCKETS = 1000
_B = 16384
_H = 50
_D = 64
_W = _D // 2
_L = 16
_NC = 2
_NS = 16
_NW = _NC * _NS
_BPW = _B // _NW
_CH = 128
_NCH = _BPW // _CH

_LO = [w if w < _L else _L + w for w in range(_W)]
_HI = [w + _L if w < _L else 2 * _L + w for w in range(_W)]


def _tec_body(idxT_hbm, qtab_hbm, rtab_hbm, outT_hbm,
              qtab_v, rtab_v, qT_v, rT_v, out_v):
    wid = lax.axis_index("s") * _NC + lax.axis_index("c")

    pltpu.sync_copy(qtab_hbm, qtab_v)
    pltpu.sync_copy(rtab_hbm, rtab_v)

    inv = jnp.float32(1.0 / _NUM_BUCKETS)
    zeros = jnp.zeros((_L,), jnp.float32)

    def g_body(g, _):
        base = wid * _BPW + g * _CH
        pltpu.sync_copy(idxT_hbm.at[:, pl.ds(base, _CH)], qT_v)

        def qr_body(i, _):
            row = lax.shift_right_logical(i, 3)
            col = (i & 7) * _L
            x = qT_v[row, pl.ds(col, _L)]
            q0 = (x.astype(jnp.float32) * inv).astype(jnp.int32)
            r0 = x - q0 * _NUM_BUCKETS
            q = (q0 + jnp.where(r0 >= _NUM_BUCKETS, 1, 0)
                 - jnp.where(r0 < 0, 1, 0))
            qT_v[row, pl.ds(col, _L)] = q
            rT_v[row, pl.ds(col, _L)] = x - q * _NUM_BUCKETS
            return 0
        lax.fori_loop(0, _H * (_CH // _L), qr_body, 0, unroll=4)

        def b16_body(t, _):
            off = t * _L
            for wg in range(4):
                def j_body(j, acc):
                    qv = qT_v[j, pl.ds(off, _L)]
                    rv = rT_v[j, pl.ds(off, _L)]
                    new = list(acc)
                    for w in range(wg * 8, wg * 8 + 8):
                        gq = plsc.load_gather(qtab_v.at[w], [qv])
                        gr = plsc.load_gather(rtab_v.at[w], [rv])
                        t = (plsc.bitcast(gq, jnp.bfloat16)
                             + plsc.bitcast(gr, jnp.bfloat16))
                        a, b = plsc.unpack(
                            t, format=plsc.PackFormat.INTERLEAVED)
                        s = w - wg * 8
                        new[2 * s] = new[2 * s] + a
                        new[2 * s + 1] = new[2 * s + 1] + b
                    return tuple(new)
                acc = lax.fori_loop(0, _H, j_body, (zeros,) * _L,
                                    unroll=2)
                for w in range(wg * 8, wg * 8 + 8):
                    s = w - wg * 8
                    out_v[_LO[w], pl.ds(off, _L)] = acc[2 * s]
                    out_v[_HI[w], pl.ds(off, _L)] = acc[2 * s + 1]
            return 0
        lax.fori_loop(0, _CH // _L, b16_body, 0)

        pltpu.sync_copy(out_v, outT_hbm.at[:, pl.ds(base, _CH)])
        return 0
    lax.fori_loop(0, _NCH, g_body, 0)


_mesh = plsc.VectorSubcoreMesh(core_axis_name="c", subcore_axis_name="s")

_qr_bag = functools.partial(
    pl.kernel,
    mesh=_mesh,
    out_type=jax.ShapeDtypeStruct((_D, _B), jnp.float32),
    scratch_types=[
        pltpu.VMEM((_W, _NUM_BUCKETS), jnp.int32),
        pltpu.VMEM((_W, _NUM_BUCKETS), jnp.int32),
        pltpu.VMEM((_H, _CH), jnp.int32),
        pltpu.VMEM((_H, _CH), jnp.int32),
        pltpu.VMEM((_D, _CH), jnp.float32),
    ],
    compiler_params=pltpu.CompilerParams(use_tc_tiling_on_sc=False,
                                         needs_layout_passes=False),
)(_tec_body)


def _pack_table(w):
    u = lax.bitcast_convert_type(w.astype(jnp.bfloat16), jnp.uint16)
    u = u.astype(jnp.uint32)
    lo = jnp.concatenate([u[:, 0:16], u[:, 32:48]], axis=1)
    hi = jnp.concatenate([u[:, 16:32], u[:, 48:64]], axis=1)
    packed = lo | (hi << 16)
    return lax.bitcast_convert_type(packed, jnp.int32).T


def kernel(input_, quotient_embed_weight, remainder_embed_weight):
    idx_t = input_.astype(jnp.int32).T
    out_t = _qr_bag(idx_t,
                    _pack_table(quotient_embed_weight),
                    _pack_table(remainder_embed_weight))
    return out_t.T


# speedup vs baseline: 1.3583x; 1.0322x over previous
import functools

import jax
import jax.numpy as jnp
from jax import lax
from jax.experimental import pallas as pl
from jax.experimental.pallas import tpu as pltpu
from jax.experimental.pallas import tpu_sc as plsc

_NUM_BUCKETS = 1000
_B = 16384
_H = 50
_D = 64
_W = _D // 2
_L = 16
_NC = 2
_NS = 16
_NW = _NC * _NS
_BPW = _B // _NW
_CH = 128
_NCH = _BPW // _CH

_LO = [w if w < _L else _L + w for w in range(_W)]
_HI = [w + _L if w < _L else 2 * _L + w for w in range(_W)]


def _tec_body(idxT_hbm, qtab_hbm, rtab_hbm, outT_hbm,
              qtab_v, rtab_v, qT_v, rT_v, out_v):
    wid = lax.axis_index("s") * _NC + lax.axis_index("c")

    pltpu.sync_copy(qtab_hbm, qtab_v)
    pltpu.sync_copy(rtab_hbm, rtab_v)

    inv = jnp.float32(1.0 / _NUM_BUCKETS)
    zeros = jnp.zeros((_L,), jnp.float32)

    def g_body(g, _):
        base = wid * _BPW + g * _CH
        pltpu.sync_copy(idxT_hbm.at[:, pl.ds(base, _CH)], qT_v)

        def qr_body(i, _):
            row = lax.shift_right_logical(i, 3)
            col = (i & 7) * _L
            x = qT_v[row, pl.ds(col, _L)]
            q0 = (x.astype(jnp.float32) * inv).astype(jnp.int32)
            r0 = x - q0 * _NUM_BUCKETS
            q = (q0 + jnp.where(r0 >= _NUM_BUCKETS, 1, 0)
                 - jnp.where(r0 < 0, 1, 0))
            qT_v[row, pl.ds(col, _L)] = q
            rT_v[row, pl.ds(col, _L)] = x - q * _NUM_BUCKETS
            return 0
        lax.fori_loop(0, _H * (_CH // _L), qr_body, 0, unroll=4)

        def b16_body(t, _):
            off = t * _L
            for wg in range(2):
                def j_body(j, acc):
                    qv = qT_v[j, pl.ds(off, _L)]
                    rv = rT_v[j, pl.ds(off, _L)]
                    new = list(acc)
                    for w in range(wg * _L, wg * _L + _L):
                        gq = plsc.load_gather(qtab_v.at[w], [qv])
                        gr = plsc.load_gather(rtab_v.at[w], [rv])
                        t = (plsc.bitcast(gq, jnp.bfloat16)
                             + plsc.bitcast(gr, jnp.bfloat16))
                        a, b = plsc.unpack(
                            t, format=plsc.PackFormat.INTERLEAVED)
                        s = w - wg * _L
                        new[2 * s] = new[2 * s] + a
                        new[2 * s + 1] = new[2 * s + 1] + b
                    return tuple(new)
                acc = lax.fori_loop(0, _H, j_body, (zeros,) * (2 * _L))
                for w in range(wg * _L, wg * _L + _L):
                    s = w - wg * _L
                    out_v[_LO[w], pl.ds(off, _L)] = acc[2 * s]
                    out_v[_HI[w], pl.ds(off, _L)] = acc[2 * s + 1]
            return 0
        lax.fori_loop(0, _CH // _L, b16_body, 0)

        pltpu.sync_copy(out_v, outT_hbm.at[:, pl.ds(base, _CH)])
        return 0
    lax.fori_loop(0, _NCH, g_body, 0)


_mesh = plsc.VectorSubcoreMesh(core_axis_name="c", subcore_axis_name="s")

_qr_bag = functools.partial(
    pl.kernel,
    mesh=_mesh,
    out_type=jax.ShapeDtypeStruct((_D, _B), jnp.float32),
    scratch_types=[
        pltpu.VMEM((_W, _NUM_BUCKETS), jnp.int32),
        pltpu.VMEM((_W, _NUM_BUCKETS), jnp.int32),
        pltpu.VMEM((_H, _CH), jnp.int32),
        pltpu.VMEM((_H, _CH), jnp.int32),
        pltpu.VMEM((_D, _CH), jnp.float32),
    ],
    compiler_params=pltpu.CompilerParams(use_tc_tiling_on_sc=False,
                                         needs_layout_passes=False),
)(_tec_body)


def _pack_table(w):
    u = lax.bitcast_convert_type(w.astype(jnp.bfloat16), jnp.uint16)
    u = u.astype(jnp.uint32)
    lo = jnp.concatenate([u[:, 0:16], u[:, 32:48]], axis=1)
    hi = jnp.concatenate([u[:, 16:32], u[:, 48:64]], axis=1)
    packed = lo | (hi << 16)
    return lax.bitcast_convert_type(packed, jnp.int32).T


def kernel(input_, quotient_embed_weight, remainder_embed_weight):
    idx_t = input_.astype(jnp.int32).T
    out_t = _qr_bag(idx_t,
                    _pack_table(quotient_embed_weight),
                    _pack_table(remainder_embed_weight))
    return out_t.T
